# baseline (speedup 1.0000x reference)
import functools

import jax
import jax.numpy as jnp
from jax import lax
from jax.experimental import pallas as pl
from jax.experimental.pallas import tpu as pltpu
from jax.experimental.pallas import tpu_sc as plsc

N = 50000
D = 128
E = 500000
NPAD = 50176
BLK = 400

ECHUNK = 2000
NCHUNKS = E // ECHUNK
NW = 32


def _deg_body(from_hbm, to_hbm, out_hbm, histf, histt, fbuf, tbuf):
    c = lax.axis_index("c")
    s = lax.axis_index("s")
    wid = s * 2 + c
    zero = jnp.zeros((16,), jnp.float32)

    def zloop(i, _):
        histf[pl.ds(i * 16, 16)] = zero
        histt[pl.ds(i * 16, 16)] = zero
        return 0

    lax.fori_loop(0, NPAD // 16, zloop, 0)

    ones = jnp.ones((16,), jnp.float32)
    nchunk = jnp.where(wid < NCHUNKS - 32 * (NCHUNKS // 32), NCHUNKS // 32 + 1,
                       NCHUNKS // 32)

    def chunk_body(ci, _):
        off = (wid + 32 * ci) * ECHUNK
        pltpu.sync_copy(from_hbm.at[pl.ds(off, ECHUNK)], fbuf)
        pltpu.sync_copy(to_hbm.at[pl.ds(off, ECHUNK)], tbuf)

        def gbody(g, _):
            fi = fbuf[pl.ds(g * 16, 16)]
            ti = tbuf[pl.ds(g * 16, 16)]
            plsc.addupdate_scatter(histf, [fi], ones)
            plsc.addupdate_scatter(histt, [ti], ones)
            return 0

        lax.fori_loop(0, ECHUNK // 16, gbody, 0)
        return 0

    lax.fori_loop(0, nchunk, chunk_body, 0)
    pltpu.sync_copy(histf, out_hbm.at[wid, 0])
    pltpu.sync_copy(histt, out_hbm.at[wid, 1])


def _degrees_sc(from_, to_):
    mesh = plsc.VectorSubcoreMesh(core_axis_name="c", subcore_axis_name="s")
    return pl.kernel(
        _deg_body,
        mesh=mesh,
        compiler_params=pltpu.CompilerParams(needs_layout_passes=False),
        out_type=jax.ShapeDtypeStruct((NW, 2, NPAD), jnp.float32),
        scratch_types=[
            pltpu.VMEM((NPAD,), jnp.float32),
            pltpu.VMEM((NPAD,), jnp.float32),
            pltpu.VMEM((ECHUNK,), jnp.int32),
            pltpu.VMEM((ECHUNK,), jnp.int32),
        ],
    )(from_, to_)


DSUM_BLK = 6272


def _dinv_body(p_ref, o_ref):
    d = jnp.sum(p_ref[...], axis=0)
    o_ref[...] = jnp.where(d > 0, jax.lax.rsqrt(d), 0.0)


def _dinv_tc(deg_partial):
    flat = deg_partial.reshape(NW, 2 * NPAD)
    out = pl.pallas_call(
        _dinv_body,
        out_shape=jax.ShapeDtypeStruct((2 * NPAD,), jnp.float32),
    )(flat)
    return out.reshape(2, NPAD)


def _scale_body(x_ref, y_ref, dx_ref, dy_ref, xs_ref, ys_ref):
    xs_ref[...] = x_ref[...] * dx_ref[...]
    ys_ref[...] = y_ref[...] * dy_ref[...]


def _prescale_tc(x, y, dxi_col, dyi_col):
    return pl.pallas_call(
        _scale_body,
        grid=(N // BLK,),
        in_specs=[
            pl.BlockSpec((BLK, D), lambda i: (i, 0)),
            pl.BlockSpec((BLK, D), lambda i: (i, 0)),
            pl.BlockSpec((BLK, 1), lambda i: (i, 0)),
            pl.BlockSpec((BLK, 1), lambda i: (i, 0)),
        ],
        out_specs=[
            pl.BlockSpec((BLK, D), lambda i: (i, 0)),
            pl.BlockSpec((BLK, D), lambda i: (i, 0)),
        ],
        out_shape=[
            jax.ShapeDtypeStruct((N, D), jnp.float32),
            jax.ShapeDtypeStruct((N, D), jnp.float32),
        ],
    )(x, y, dxi_col, dyi_col)


def _residual_body(x_ref, y_ref, s1_ref, s2_ref, dx_ref, dy_ref,
                   ru_ref, ri_ref):
    ru_ref[...] = (x_ref[...] + dx_ref[...] * s1_ref[...]) * 0.5
    ri_ref[...] = (y_ref[...] + dy_ref[...] * s2_ref[...]) * 0.5


def _residual_tc(x, y, s1_pad, s2_pad, dxi_col, dyi_col):
    return pl.pallas_call(
        _residual_body,
        grid=(N // BLK,),
        in_specs=[
            pl.BlockSpec((BLK, D), lambda i: (i, 0)),
            pl.BlockSpec((BLK, D), lambda i: (i, 0)),
            pl.BlockSpec((BLK, D), lambda i: (i, 0)),
            pl.BlockSpec((BLK, D), lambda i: (i, 0)),
            pl.BlockSpec((BLK, 1), lambda i: (i, 0)),
            pl.BlockSpec((BLK, 1), lambda i: (i, 0)),
        ],
        out_specs=[
            pl.BlockSpec((BLK, D), lambda i: (i, 0)),
            pl.BlockSpec((BLK, D), lambda i: (i, 0)),
        ],
        out_shape=[
            jax.ShapeDtypeStruct((N, D), jnp.float32),
            jax.ShapeDtypeStruct((N, D), jnp.float32),
        ],
    )(x, y, s1_pad, s2_pad, dxi_col, dyi_col)


CH = 6272
NCH = 8
ACC_ROWS = 6288
DUMMY = 6272
STRIPE = CH // 16
RING = 64
BATCH = 64
BSHIFT = 6
ZROWS = 16
PCHUNK = 2000
PNCH = E // PCHUNK
UNROLL = 25


def _prop_body(from_hbm, to_hbm, xs_hbm, ys_hbm, s2_hbm, s1_hbm,
               acc, sel_src, sel_dst, ebuf_f, ebuf_t, rbuf, zbuf,
               sem_g, sem_s, sem_e):
    c = lax.axis_index("c")
    s = lax.axis_index("s")
    zero16 = jnp.zeros((16,), jnp.float32)
    lane = jax.lax.iota(jnp.int32, 16)

    def zb(r, _):
        for k in range(D // 16):
            zbuf[r, pl.ds(k * 16, 16)] = zero16
        return 0
    lax.fori_loop(0, ZROWS, zb, 0)

    nchunk = jnp.where(s < PNCH - 16 * (PNCH // 16), PNCH // 16 + 1,
                       PNCH // 16)

    def issue_edges(ci):
        sbase = lax.bitwise_and(ci, 1) * PCHUNK
        off = (s + 16 * ci) * PCHUNK
        pltpu.async_copy(from_hbm.at[pl.ds(off, PCHUNK)],
                         ebuf_f.at[pl.ds(sbase, PCHUNK)], sem_e)
        pltpu.async_copy(to_hbm.at[pl.ds(off, PCHUNK)],
                         ebuf_t.at[pl.ds(sbase, PCHUNK)], sem_e)

    def wait_edges():
        pltpu.make_async_copy(from_hbm.at[pl.ds(0, PCHUNK)],
                              ebuf_f.at[pl.ds(0, PCHUNK)], sem_e).wait()
        pltpu.make_async_copy(to_hbm.at[pl.ds(0, PCHUNK)],
                              ebuf_t.at[pl.ds(0, PCHUNK)], sem_e).wait()

    def do_job(direction, tab_hbm, out_hbm, lo):
        zbase = s * STRIPE
        hs = []
        for i in range(STRIPE // ZROWS):
            hs.append(pltpu.async_copy(
                zbuf, acc.at[pl.ds(zbase + i * ZROWS, ZROWS)], sem_s))
        hs.append(pltpu.async_copy(
            zbuf.at[pl.ds(0, STRIPE % ZROWS)],
            acc.at[pl.ds(zbase + (STRIPE // ZROWS) * ZROWS, STRIPE % ZROWS)],
            sem_s))
        for h in hs:
            h.wait()
        plsc.subcore_barrier()

        def wait_gather():
            pltpu.make_async_copy(tab_hbm.at[sel_src.at[0]], rbuf.at[0],
                                  sem_g).wait()

        def wait_scatter():
            pltpu.make_async_copy(rbuf.at[0], acc.at[sel_dst.at[0]],
                                  sem_s).wait()

        def issue_scatter(b):
            slot = lax.bitwise_and(b, 1)
            r = lax.bitwise_and(b, RING - 1)
            pltpu.async_copy(rbuf.at[slot], acc.at[sel_dst.at[r]], sem_s,
                             add=True)

        def flush_step(b, _):
            slot = lax.bitwise_and(b, 1)
            r = lax.bitwise_and(b, RING - 1)

            @pl.when(b >= 2)
            def _():
                wait_scatter()

            pltpu.async_copy(tab_hbm.at[sel_src.at[r]], rbuf.at[slot], sem_g)

            @pl.when(b >= 1)
            def _():
                wait_gather()
                issue_scatter(b - 1)

            return 0

        issue_edges(0)

        def chunk_body(ci, carry):
            wp, fb = carry
            wait_edges()

            @pl.when(ci + 1 < nchunk)
            def _():
                issue_edges(ci + 1)

            sbase = lax.bitwise_and(ci, 1) * PCHUNK
            dst_buf = ebuf_t if direction == 0 else ebuf_f
            src_buf = ebuf_f if direction == 0 else ebuf_t

            def group(g, wp):
                subs = []
                for u in range(UNROLL):
                    dst = dst_buf[pl.ds(sbase + (g * UNROLL + u) * 16, 16)]
                    src = src_buf[pl.ds(sbase + (g * UNROLL + u) * 16, 16)]
                    m = (dst >= lo) & (dst < lo + CH)
                    mi = jnp.where(m, 1, 0).astype(jnp.int32)
                    excl = plsc.cumsum(mi) - mi
                    cnt = plsc.all_reduce_population_count(m)
                    subs.append((dst, src, m, excl, cnt))
                for dst, src, m, excl, cnt in subs:
                    off_v = wp + excl
                    row = lax.bitwise_and(
                        lax.shift_right_logical(off_v, BSHIFT), RING - 1)
                    col = lax.bitwise_and(off_v, BATCH - 1)
                    plsc.store_scatter(sel_src, [row, col], src, mask=m)
                    plsc.store_scatter(sel_dst, [row, col], dst - lo, mask=m)
                    wp = wp + cnt
                return wp

            wp = lax.fori_loop(0, PCHUNK // (16 * UNROLL), group, wp)
            nb = lax.shift_right_logical(jnp.max(wp), BSHIFT)
            lax.fori_loop(fb, nb, flush_step, 0)
            return (wp, nb)

        wp, fb = lax.fori_loop(
            0, nchunk, chunk_body, (jnp.zeros((16,), jnp.int32), jnp.int32(0)))

        n_sel = jnp.max(wp)
        n_pad = lax.bitwise_and(n_sel + (BATCH - 1), -BATCH)
        for i in range(BATCH // 16):
            off_v = n_sel + i * 16 + lane
            m = off_v < n_pad
            row = lax.bitwise_and(lax.shift_right_logical(off_v, BSHIFT), RING - 1)
            col = lax.bitwise_and(off_v, BATCH - 1)
            plsc.store_scatter(sel_src, [row, col], jnp.zeros((16,), jnp.int32),
                               mask=m)
            plsc.store_scatter(sel_dst, [row, col],
                               jnp.full((16,), DUMMY, jnp.int32), mask=m)
        nb_all = lax.shift_right_logical(n_pad, BSHIFT)
        lax.fori_loop(fb, nb_all, flush_step, 0)

        @pl.when(nb_all >= 1)
        def _():
            wait_gather()
            issue_scatter(nb_all - 1)

        @pl.when(nb_all >= 2)
        def _():
            wait_scatter()

        @pl.when(nb_all >= 1)
        def _():
            wait_scatter()

        plsc.subcore_barrier()

        ob = s * STRIPE
        hs = []
        for i in range(STRIPE // BATCH):
            hs.append(pltpu.async_copy(
                acc.at[pl.ds(ob + i * BATCH, BATCH)],
                out_hbm.at[pl.ds(lo + ob + i * BATCH, BATCH)], sem_g))
        hs.append(pltpu.async_copy(
            acc.at[pl.ds(ob + (STRIPE // BATCH) * BATCH, STRIPE % BATCH)],
            out_hbm.at[pl.ds(lo + ob + (STRIPE // BATCH) * BATCH,
                             STRIPE % BATCH)], sem_g))
        for h in hs:
            h.wait()
        plsc.subcore_barrier()

    for direction in range(2):
        tab_hbm = xs_hbm if direction == 0 else ys_hbm
        out_hbm = s2_hbm if direction == 0 else s1_hbm
        for j in range(NCH // 2):
            do_job(direction, tab_hbm, out_hbm, (c + 2 * j) * CH)


def _propagate_sc(from_, to_, xs, ys):
    mesh = plsc.VectorSubcoreMesh(core_axis_name="c", subcore_axis_name="s")
    return pl.kernel(
        _prop_body,
        mesh=mesh,
        compiler_params=pltpu.CompilerParams(needs_layout_passes=False),
        out_type=(
            jax.ShapeDtypeStruct((NCH * CH, D), jnp.float32),
            jax.ShapeDtypeStruct((NCH * CH, D), jnp.float32),
        ),
        scratch_types=[
            pltpu.VMEM_SHARED((ACC_ROWS, D), jnp.float32),
            pltpu.VMEM((RING, BATCH), jnp.int32),
            pltpu.VMEM((RING, BATCH), jnp.int32),
            pltpu.VMEM((2 * PCHUNK,), jnp.int32),
            pltpu.VMEM((2 * PCHUNK,), jnp.int32),
            pltpu.VMEM((2, BATCH, D), jnp.float32),
            pltpu.VMEM((ZROWS, D), jnp.float32),
            pltpu.SemaphoreType.DMA,
            pltpu.SemaphoreType.DMA,
            pltpu.SemaphoreType.DMA,
        ],
    )(from_, to_, xs, ys)


def kernel(user_table, item_table, user_node_id, item_node_id, edge_index):
    x = user_table
    y = item_table
    e = edge_index.astype(jnp.int32)
    from_, to_ = e[0], e[1]

    deg_partial = _degrees_sc(from_, to_)
    dinv = _dinv_tc(deg_partial)
    dxi_col = dinv[0][:, None]
    dyi_col = dinv[1][:, None]

    xs, ys = _prescale_tc(x, y, dxi_col, dyi_col)
    s2_pad, s1_pad = _propagate_sc(from_, to_, xs, ys)
    res_user, res_item = _residual_tc(x, y, s1_pad, s2_pad, dxi_col, dyi_col)
    return (res_user, res_item)


# speedup vs baseline: 1.0451x; 1.0451x over previous
import functools

import jax
import jax.numpy as jnp
from jax import lax
from jax.experimental import pallas as pl
from jax.experimental.pallas import tpu as pltpu
from jax.experimental.pallas import tpu_sc as plsc

N = 50000
D = 128
E = 500000
NPAD = 50176
BLK = 400

ECHUNK = 2000
NCHUNKS = E // ECHUNK
NW = 32


def _deg_body(from_hbm, to_hbm, out_hbm, histf, histt, fbuf, tbuf):
    c = lax.axis_index("c")
    s = lax.axis_index("s")
    wid = s * 2 + c
    zero = jnp.zeros((16,), jnp.float32)

    def zloop(i, _):
        histf[pl.ds(i * 16, 16)] = zero
        histt[pl.ds(i * 16, 16)] = zero
        return 0

    lax.fori_loop(0, NPAD // 16, zloop, 0)

    ones = jnp.ones((16,), jnp.float32)
    nchunk = jnp.where(wid < NCHUNKS - 32 * (NCHUNKS // 32), NCHUNKS // 32 + 1,
                       NCHUNKS // 32)

    def chunk_body(ci, _):
        off = (wid + 32 * ci) * ECHUNK
        pltpu.sync_copy(from_hbm.at[pl.ds(off, ECHUNK)], fbuf)
        pltpu.sync_copy(to_hbm.at[pl.ds(off, ECHUNK)], tbuf)

        def gbody(g, _):
            fi = fbuf[pl.ds(g * 16, 16)]
            ti = tbuf[pl.ds(g * 16, 16)]
            plsc.addupdate_scatter(histf, [fi], ones)
            plsc.addupdate_scatter(histt, [ti], ones)
            return 0

        lax.fori_loop(0, ECHUNK // 16, gbody, 0)
        return 0

    lax.fori_loop(0, nchunk, chunk_body, 0)
    pltpu.sync_copy(histf, out_hbm.at[wid, 0])
    pltpu.sync_copy(histt, out_hbm.at[wid, 1])


def _degrees_sc(from_, to_):
    mesh = plsc.VectorSubcoreMesh(core_axis_name="c", subcore_axis_name="s")
    return pl.kernel(
        _deg_body,
        mesh=mesh,
        compiler_params=pltpu.CompilerParams(needs_layout_passes=False),
        out_type=jax.ShapeDtypeStruct((NW, 2, NPAD), jnp.float32),
        scratch_types=[
            pltpu.VMEM((NPAD,), jnp.float32),
            pltpu.VMEM((NPAD,), jnp.float32),
            pltpu.VMEM((ECHUNK,), jnp.int32),
            pltpu.VMEM((ECHUNK,), jnp.int32),
        ],
    )(from_, to_)


DSUM_BLK = 6272


def _dinv_body(p_ref, o_ref):
    d = jnp.sum(p_ref[...], axis=0)
    o_ref[...] = jnp.where(d > 0, jax.lax.rsqrt(d), 0.0)


def _dinv_tc(deg_partial):
    flat = deg_partial.reshape(NW, 2 * NPAD)
    out = pl.pallas_call(
        _dinv_body,
        out_shape=jax.ShapeDtypeStruct((2 * NPAD,), jnp.float32),
    )(flat)
    return out.reshape(2, NPAD)


def _scale_body(x_ref, y_ref, dx_ref, dy_ref, xs_ref, ys_ref):
    xs_ref[...] = x_ref[...] * dx_ref[...]
    ys_ref[...] = y_ref[...] * dy_ref[...]


def _prescale_tc(x, y, dxi_col, dyi_col):
    return pl.pallas_call(
        _scale_body,
        grid=(N // BLK,),
        in_specs=[
            pl.BlockSpec((BLK, D), lambda i: (i, 0)),
            pl.BlockSpec((BLK, D), lambda i: (i, 0)),
            pl.BlockSpec((BLK, 1), lambda i: (i, 0)),
            pl.BlockSpec((BLK, 1), lambda i: (i, 0)),
        ],
        out_specs=[
            pl.BlockSpec((BLK, D), lambda i: (i, 0)),
            pl.BlockSpec((BLK, D), lambda i: (i, 0)),
        ],
        out_shape=[
            jax.ShapeDtypeStruct((N, D), jnp.float32),
            jax.ShapeDtypeStruct((N, D), jnp.float32),
        ],
    )(x, y, dxi_col, dyi_col)


def _residual_body(x_ref, y_ref, s1_ref, s2_ref, dx_ref, dy_ref,
                   ru_ref, ri_ref):
    ru_ref[...] = (x_ref[...] + dx_ref[...] * s1_ref[...]) * 0.5
    ri_ref[...] = (y_ref[...] + dy_ref[...] * s2_ref[...]) * 0.5


def _residual_tc(x, y, s1_pad, s2_pad, dxi_col, dyi_col):
    return pl.pallas_call(
        _residual_body,
        grid=(N // BLK,),
        in_specs=[
            pl.BlockSpec((BLK, D), lambda i: (i, 0)),
            pl.BlockSpec((BLK, D), lambda i: (i, 0)),
            pl.BlockSpec((BLK, D), lambda i: (i, 0)),
            pl.BlockSpec((BLK, D), lambda i: (i, 0)),
            pl.BlockSpec((BLK, 1), lambda i: (i, 0)),
            pl.BlockSpec((BLK, 1), lambda i: (i, 0)),
        ],
        out_specs=[
            pl.BlockSpec((BLK, D), lambda i: (i, 0)),
            pl.BlockSpec((BLK, D), lambda i: (i, 0)),
        ],
        out_shape=[
            jax.ShapeDtypeStruct((N, D), jnp.float32),
            jax.ShapeDtypeStruct((N, D), jnp.float32),
        ],
    )(x, y, s1_pad, s2_pad, dxi_col, dyi_col)


CH = 6272
NCH = 8
ACC_ROWS = 6288
DUMMY = 6272
STRIPE = CH // 16
RING = 64
BATCH = 64
BSHIFT = 6
ZROWS = 16
PCHUNK = 2000
PNCH = E // PCHUNK
UNROLL = 5


def _prop_body(from_hbm, to_hbm, xs_hbm, ys_hbm, s2_hbm, s1_hbm,
               acc, sel_src, sel_dst, ebuf_f, ebuf_t, rbuf, zbuf,
               sem_g, sem_s, sem_e):
    c = lax.axis_index("c")
    s = lax.axis_index("s")
    zero16 = jnp.zeros((16,), jnp.float32)
    lane = jax.lax.iota(jnp.int32, 16)

    def zb(r, _):
        for k in range(D // 16):
            zbuf[r, pl.ds(k * 16, 16)] = zero16
        return 0
    lax.fori_loop(0, ZROWS, zb, 0)

    nchunk = jnp.where(s < PNCH - 16 * (PNCH // 16), PNCH // 16 + 1,
                       PNCH // 16)

    def issue_edges(ci):
        sbase = lax.bitwise_and(ci, 1) * PCHUNK
        off = (s + 16 * ci) * PCHUNK
        pltpu.async_copy(from_hbm.at[pl.ds(off, PCHUNK)],
                         ebuf_f.at[pl.ds(sbase, PCHUNK)], sem_e)
        pltpu.async_copy(to_hbm.at[pl.ds(off, PCHUNK)],
                         ebuf_t.at[pl.ds(sbase, PCHUNK)], sem_e)

    def wait_edges():
        pltpu.make_async_copy(from_hbm.at[pl.ds(0, PCHUNK)],
                              ebuf_f.at[pl.ds(0, PCHUNK)], sem_e).wait()
        pltpu.make_async_copy(to_hbm.at[pl.ds(0, PCHUNK)],
                              ebuf_t.at[pl.ds(0, PCHUNK)], sem_e).wait()

    def do_job(direction, tab_hbm, out_hbm, lo):
        zbase = s * STRIPE
        hs = []
        for i in range(STRIPE // ZROWS):
            hs.append(pltpu.async_copy(
                zbuf, acc.at[pl.ds(zbase + i * ZROWS, ZROWS)], sem_s))
        hs.append(pltpu.async_copy(
            zbuf.at[pl.ds(0, STRIPE % ZROWS)],
            acc.at[pl.ds(zbase + (STRIPE // ZROWS) * ZROWS, STRIPE % ZROWS)],
            sem_s))
        for h in hs:
            h.wait()
        plsc.subcore_barrier()

        def wait_gather():
            pltpu.make_async_copy(tab_hbm.at[sel_src.at[0]], rbuf.at[0],
                                  sem_g).wait()

        def wait_scatter():
            pltpu.make_async_copy(rbuf.at[0], acc.at[sel_dst.at[0]],
                                  sem_s).wait()

        def issue_scatter(b):
            slot = lax.rem(b, 3)
            r = lax.bitwise_and(b, RING - 1)
            pltpu.async_copy(rbuf.at[slot], acc.at[sel_dst.at[r]], sem_s,
                             add=True)

        def flush_step(b, _):
            slot = lax.rem(b, 3)
            r = lax.bitwise_and(b, RING - 1)

            @pl.when(b >= 3)
            def _():
                wait_scatter()

            pltpu.async_copy(tab_hbm.at[sel_src.at[r]], rbuf.at[slot], sem_g)

            @pl.when(b >= 1)
            def _():
                wait_gather()
                issue_scatter(b - 1)

            return 0

        issue_edges(0)

        def chunk_body(ci, carry):
            wp, fb = carry
            wait_edges()

            @pl.when(ci + 1 < nchunk)
            def _():
                issue_edges(ci + 1)

            sbase = lax.bitwise_and(ci, 1) * PCHUNK
            dst_buf = ebuf_t if direction == 0 else ebuf_f
            src_buf = ebuf_f if direction == 0 else ebuf_t

            def group(g, wp):
                subs = []
                for u in range(UNROLL):
                    dst = dst_buf[pl.ds(sbase + (g * UNROLL + u) * 16, 16)]
                    src = src_buf[pl.ds(sbase + (g * UNROLL + u) * 16, 16)]
                    m = (dst >= lo) & (dst < lo + CH)
                    mi = jnp.where(m, 1, 0).astype(jnp.int32)
                    excl = plsc.cumsum(mi) - mi
                    cnt = plsc.all_reduce_population_count(m)
                    subs.append((dst, src, m, excl, cnt))
                for dst, src, m, excl, cnt in subs:
                    off_v = wp + excl
                    row = lax.bitwise_and(
                        lax.shift_right_logical(off_v, BSHIFT), RING - 1)
                    col = lax.bitwise_and(off_v, BATCH - 1)
                    plsc.store_scatter(sel_src, [row, col], src, mask=m)
                    plsc.store_scatter(sel_dst, [row, col], dst - lo, mask=m)
                    wp = wp + cnt
                return wp

            wp = lax.fori_loop(0, PCHUNK // (16 * UNROLL), group, wp)
            nb = lax.shift_right_logical(jnp.max(wp), BSHIFT)
            lax.fori_loop(fb, nb, flush_step, 0)
            return (wp, nb)

        wp, fb = lax.fori_loop(
            0, nchunk, chunk_body, (jnp.zeros((16,), jnp.int32), jnp.int32(0)))

        n_sel = jnp.max(wp)
        n_pad = lax.bitwise_and(n_sel + (BATCH - 1), -BATCH)
        for i in range(BATCH // 16):
            off_v = n_sel + i * 16 + lane
            m = off_v < n_pad
            row = lax.bitwise_and(lax.shift_right_logical(off_v, BSHIFT), RING - 1)
            col = lax.bitwise_and(off_v, BATCH - 1)
            plsc.store_scatter(sel_src, [row, col], jnp.zeros((16,), jnp.int32),
                               mask=m)
            plsc.store_scatter(sel_dst, [row, col],
                               jnp.full((16,), DUMMY, jnp.int32), mask=m)
        nb_all = lax.shift_right_logical(n_pad, BSHIFT)
        lax.fori_loop(fb, nb_all, flush_step, 0)

        @pl.when(nb_all >= 1)
        def _():
            wait_gather()
            issue_scatter(nb_all - 1)

        @pl.when(nb_all >= 3)
        def _():
            wait_scatter()

        @pl.when(nb_all >= 2)
        def _():
            wait_scatter()

        @pl.when(nb_all >= 1)
        def _():
            wait_scatter()

        plsc.subcore_barrier()

        ob = s * STRIPE
        hs = []
        for i in range(STRIPE // BATCH):
            hs.append(pltpu.async_copy(
                acc.at[pl.ds(ob + i * BATCH, BATCH)],
                out_hbm.at[pl.ds(lo + ob + i * BATCH, BATCH)], sem_g))
        hs.append(pltpu.async_copy(
            acc.at[pl.ds(ob + (STRIPE // BATCH) * BATCH, STRIPE % BATCH)],
            out_hbm.at[pl.ds(lo + ob + (STRIPE // BATCH) * BATCH,
                             STRIPE % BATCH)], sem_g))
        for h in hs:
            h.wait()
        plsc.subcore_barrier()

    for direction in range(2):
        tab_hbm = xs_hbm if direction == 0 else ys_hbm
        out_hbm = s2_hbm if direction == 0 else s1_hbm
        for j in range(NCH // 2):
            do_job(direction, tab_hbm, out_hbm, (c + 2 * j) * CH)


def _propagate_sc(from_, to_, xs, ys):
    mesh = plsc.VectorSubcoreMesh(core_axis_name="c", subcore_axis_name="s")
    return pl.kernel(
        _prop_body,
        mesh=mesh,
        compiler_params=pltpu.CompilerParams(needs_layout_passes=False),
        out_type=(
            jax.ShapeDtypeStruct((NCH * CH, D), jnp.float32),
            jax.ShapeDtypeStruct((NCH * CH, D), jnp.float32),
        ),
        scratch_types=[
            pltpu.VMEM_SHARED((ACC_ROWS, D), jnp.float32),
            pltpu.VMEM((RING, BATCH), jnp.int32),
            pltpu.VMEM((RING, BATCH), jnp.int32),
            pltpu.VMEM((2 * PCHUNK,), jnp.int32),
            pltpu.VMEM((2 * PCHUNK,), jnp.int32),
            pltpu.VMEM((3, BATCH, D), jnp.float32),
            pltpu.VMEM((ZROWS, D), jnp.float32),
            pltpu.SemaphoreType.DMA,
            pltpu.SemaphoreType.DMA,
            pltpu.SemaphoreType.DMA,
        ],
    )(from_, to_, xs, ys)


def kernel(user_table, item_table, user_node_id, item_node_id, edge_index):
    x = user_table
    y = item_table
    e = edge_index.astype(jnp.int32)
    from_, to_ = e[0], e[1]

    deg_partial = _degrees_sc(from_, to_)
    dinv = _dinv_tc(deg_partial)
    dxi_col = dinv[0][:, None]
    dyi_col = dinv[1][:, None]

    xs, ys = _prescale_tc(x, y, dxi_col, dyi_col)
    s2_pad, s1_pad = _propagate_sc(from_, to_, xs, ys)
    res_user, res_item = _residual_tc(x, y, s1_pad, s2_pad, dxi_col, dyi_col)
    return (res_user, res_item)
